# Initial kernel scaffold; baseline (speedup 1.0000x reference)
import jax
import jax.numpy as jnp
from jax.experimental import pallas as pl


def kernel(x, E_w):
    raise NotImplementedError("write your pallas kernel here")


# speedup vs baseline: 9.2456x; 9.2456x over previous
import functools

import jax
import jax.numpy as jnp
from jax import lax
from jax.experimental import pallas as pl
from jax.experimental.pallas import tpu as pltpu
from jax.experimental.pallas import tpu_sc as plsc

D_MODEL = 128

_NC = 2
_NS = 16
_NW = _NC * _NS

_CH = 128


def _make_gather(n_rows: int):
    assert n_rows % _NW == 0
    b_per_w = n_rows // _NW
    assert b_per_w % _CH == 0
    nch = b_per_w // _CH
    nbuf = 5 if nch % 5 == 0 else (4 if nch % 4 == 0 else 2)
    assert nch % nbuf == 0
    ngrp = nch // nbuf

    mesh = plsc.VectorSubcoreMesh(core_axis_name="c", subcore_axis_name="s")

    @functools.partial(
        pl.kernel,
        mesh=mesh,
        out_type=jax.ShapeDtypeStruct((n_rows, D_MODEL), jnp.float32),
        scratch_types=[
            pltpu.VMEM((nch, _CH), jnp.int32),
            pltpu.VMEM((nbuf, _CH, D_MODEL), jnp.float32),
        ]
        + [pltpu.SemaphoreType.DMA] * (2 * nbuf),
    )
    def gather_kernel(idx_hbm, table_hbm, out_hbm, idx_v, rows_v, *sems):
        gsem = sems[:nbuf]
        wsem = sems[nbuf:]
        wid = lax.axis_index("s") * _NC + lax.axis_index("c")

        pltpu.sync_copy(idx_hbm.at[wid], idx_v)

        out_base = wid * b_per_w

        def start_gather(c, b):
            pltpu.async_copy(table_hbm.at[idx_v.at[c]], rows_v.at[b], gsem[b])

        def wait_gather(c, b):
            pltpu.make_async_copy(
                table_hbm.at[idx_v.at[c]], rows_v.at[b], gsem[b]
            ).wait()

        def start_write(c, b):
            pltpu.async_copy(
                rows_v.at[b], out_hbm.at[pl.ds(out_base + c * _CH, _CH)], wsem[b]
            )

        def wait_write(c, b):
            pltpu.make_async_copy(
                rows_v.at[b], out_hbm.at[pl.ds(out_base + c * _CH, _CH)], wsem[b]
            ).wait()

        for b in range(nbuf):
            start_gather(b, b)

        def body(g, carry):
            c0 = g * nbuf
            for b in range(nbuf):
                c = c0 + b
                wait_gather(c, b)
                start_write(c, b)
                wait_write(c, b)
                start_gather(c + nbuf, b)
            return carry

        lax.fori_loop(0, ngrp - 1, body, 0)

        c0 = (ngrp - 1) * nbuf
        for b in range(nbuf):
            c = c0 + b
            wait_gather(c, b)
            start_write(c, b)
            wait_write(c, b)

    return gather_kernel


def kernel(x, E_w):
    B, S = x.shape
    n = B * S
    b_per_w = n // _NW
    nch = b_per_w // _CH
    idx = x.astype(jnp.int32).reshape(_NW, nch, _CH)
    out = _make_gather(n)(idx, E_w)
    return out.reshape(B, S, D_MODEL)
